# baseline (speedup 1.0000x reference)
import functools

import jax
import jax.numpy as jnp
from jax import lax
from jax.experimental import pallas as pl
from jax.experimental.pallas import tpu as pltpu
from jax.experimental.pallas import tpu_sc as plsc

N = 50000
E = 800000
PE = 100000
D = 128
HID = 72

EPAD = 819200
EROWS = 6400
EROWS64 = 12800
PEPAD = 131072
PEROWS = 1024

HALF = 25000
QUARTER = 12500
ACC4 = 12544
JUNK_Q = 12500
CAPW = 25600
CAPC = 200
DEG_ROWS = 51200
JUNK_DEG = 50432

_SC_MESH = dict(core_axis_name="c", subcore_axis_name="s")
_SC_PARAMS = pltpu.CompilerParams(use_tc_tiling_on_sc=False)
_SC_PARAMS_NL = pltpu.CompilerParams(use_tc_tiling_on_sc=False,
                                     needs_layout_passes=False)


def _deg_body(col2, zeros1d, out, colv, ones_v, acc, sem):
    c = lax.axis_index("c")
    s = lax.axis_index("s")
    wid = s * 2 + c
    pltpu.sync_copy(zeros1d, acc.at[pl.ds(s * 3200, 3200)])
    for i in range(8):
        ones_v[pl.ds(i * 16, 16)] = jnp.ones((16,), jnp.float32)
    pltpu.sync_copy(col2.at[pl.ds(wid * 200, 200)], colv)
    plsc.subcore_barrier()

    def body(j, carry):
        pltpu.sync_copy(ones_v, acc.at[colv.at[j]], add=True)
        return carry

    lax.fori_loop(0, 200, body, 0)
    plsc.subcore_barrier()
    pltpu.sync_copy(acc.at[pl.ds(s * 3200, 3200)],
                    out.at[c, pl.ds(s * 3200, 3200)])


def _make_deg_kernel():
    return functools.partial(
        pl.kernel,
        out_type=jax.ShapeDtypeStruct((2, DEG_ROWS), jnp.float32),
        mesh=plsc.VectorSubcoreMesh(**_SC_MESH),
        compiler_params=_SC_PARAMS,
        scratch_types=[
            pltpu.VMEM((200, 128), jnp.int32),
            pltpu.VMEM((128,), jnp.float32),
            pltpu.VMEM_SHARED((DEG_ROWS,), jnp.float32),
            pltpu.SemaphoreType.DMA,
        ],
    )(_deg_body)


def _part_body(row2, col2, selr, sellc, counts, rv, cv, bufr, bufl, cntv):
    c = lax.axis_index("c")
    s = lax.axis_index("s")
    w = s * 2 + c

    for b in range(4):
        base = b * QUARTER

        def blk(o, off):
            r0 = w * 400 + o * 8
            pltpu.sync_copy(row2.at[pl.ds(r0, 8)], rv)
            pltpu.sync_copy(col2.at[pl.ds(r0, 8)], cv)
            for j in range(8):
                for i in range(4):
                    rvv = rv[j, pl.ds(i * 16, 16)]
                    lc = cv[j, pl.ds(i * 16, 16)] - base
                    m = (lc >= 0) & (lc < QUARTER)
                    mi = m.astype(jnp.int32)
                    cum = plsc.cumsum(mi)
                    pos = off + cum - mi
                    plsc.store_scatter(bufr, [pos], rvv, mask=m)
                    plsc.store_scatter(bufl, [pos], lc, mask=m)
                    off = off + jnp.max(cum)
            return off

        off = lax.fori_loop(0, 50, blk, 0)
        target = ((off + 1023) // 1024) * 1024

        def padb(k, off2):
            pos = off2 + lax.iota(jnp.int32, 16)
            plsc.store_scatter(bufr, [pos], jnp.zeros((16,), jnp.int32))
            plsc.store_scatter(bufl, [pos], jnp.full((16,), JUNK_Q, jnp.int32))
            return off2 + 16

        lax.fori_loop(0, (target - off + 15) // 16, padb, off)
        pltpu.sync_copy(bufr.at[pl.ds(0, CAPW)], selr.at[b, w])
        pltpu.sync_copy(bufl.at[pl.ds(0, CAPW)], sellc.at[b, w])
        cntv[pl.ds(0, 16)] = jnp.full((16,), target // 128, jnp.int32)
        pltpu.sync_copy(cntv, counts.at[b, w])


def _make_part_kernel():
    return functools.partial(
        pl.kernel,
        out_type=(jax.ShapeDtypeStruct((4, 32, CAPW), jnp.int32),
                  jax.ShapeDtypeStruct((4, 32, CAPW), jnp.int32),
                  jax.ShapeDtypeStruct((4, 32, 16), jnp.int32)),
        mesh=plsc.VectorSubcoreMesh(**_SC_MESH),
        compiler_params=_SC_PARAMS_NL,
        scratch_types=[
            pltpu.VMEM((8, 64), jnp.int32),
            pltpu.VMEM((8, 64), jnp.int32),
            pltpu.VMEM((CAPW + 16,), jnp.int32),
            pltpu.VMEM((CAPW + 16,), jnp.int32),
            pltpu.VMEM((16,), jnp.int32),
        ],
    )(_part_body)


def _spmm_body(t_hbm, selr, sellc, counts, zeros2d, outp,
               row_v, col_v, cnt_v, rows_a, rows_b, acc, sga, sgb, ssa, ssb):
    c = lax.axis_index("c")
    s = lax.axis_index("s")

    for p in range(2):
        b = 2 * c + p
        pltpu.sync_copy(zeros2d, acc.at[pl.ds(s * 784, 784)])
        plsc.subcore_barrier()
        for q in range(2):
            w2 = s * 2 + q
            pltpu.sync_copy(counts.at[b, w2], cnt_v)
            nblk = jnp.max(cnt_v[pl.ds(0, 16)]) // 8

            def blk(b0, carry):
                pltpu.sync_copy(selr.at[b, w2, pl.ds(b0 * 8, 8)], row_v)
                pltpu.sync_copy(sellc.at[b, w2, pl.ds(b0 * 8, 8)], col_v)
                pend = {}
                for j in range(8):
                    buf, sg, ss = ((rows_a, sga, ssa) if j % 2 == 0
                                   else (rows_b, sgb, ssb))
                    if j >= 2:
                        pend[j - 2].wait()
                    g = pltpu.async_copy(t_hbm.at[row_v.at[j]], buf, sg)
                    g.wait()
                    pend[j] = pltpu.async_copy(buf, acc.at[col_v.at[j]], ss,
                                               add=True)
                pend[6].wait()
                pend[7].wait()
                return carry

            lax.fori_loop(0, nblk, blk, 0)
        plsc.subcore_barrier()
        pltpu.sync_copy(acc.at[pl.ds(s * 784, 784)],
                        outp.at[b, pl.ds(s * 784, 784)])
        plsc.subcore_barrier()


def _make_spmm_kernel():
    return functools.partial(
        pl.kernel,
        out_type=jax.ShapeDtypeStruct((4, ACC4, HID), jnp.float32),
        mesh=plsc.VectorSubcoreMesh(**_SC_MESH),
        compiler_params=_SC_PARAMS_NL,
        scratch_types=[
            pltpu.VMEM((8, 128), jnp.int32),
            pltpu.VMEM((8, 128), jnp.int32),
            pltpu.VMEM((16,), jnp.int32),
            pltpu.VMEM((128, HID), jnp.float32),
            pltpu.VMEM((128, HID), jnp.float32),
            pltpu.VMEM_SHARED((ACC4, HID), jnp.float32),
            pltpu.SemaphoreType.DMA,
            pltpu.SemaphoreType.DMA,
            pltpu.SemaphoreType.DMA,
            pltpu.SemaphoreType.DMA,
        ],
    )(_spmm_body)


def _pedge_body(u_hbm, v_hbm, src2, dst2, outU, outV, si, di, ub, vb,
                semu, semv):
    c = lax.axis_index("c")
    s = lax.axis_index("s")
    wid = s * 2 + c
    pltpu.sync_copy(src2.at[pl.ds(wid * 32, 32)], si)
    pltpu.sync_copy(dst2.at[pl.ds(wid * 32, 32)], di)

    def body(j, carry):
        du = pltpu.async_copy(u_hbm.at[si.at[j]], ub, semu)
        dv = pltpu.async_copy(v_hbm.at[di.at[j]], vb, semv)
        du.wait()
        pltpu.sync_copy(ub, outU.at[pl.ds(wid * 4096 + j * 128, 128)])
        dv.wait()
        pltpu.sync_copy(vb, outV.at[pl.ds(wid * 4096 + j * 128, 128)])
        return carry

    lax.fori_loop(0, 32, body, 0)


def _make_pedge_kernel():
    return functools.partial(
        pl.kernel,
        out_type=(jax.ShapeDtypeStruct((PEPAD, HID), jnp.float32),
                  jax.ShapeDtypeStruct((PEPAD, HID), jnp.float32)),
        mesh=plsc.VectorSubcoreMesh(**_SC_MESH),
        compiler_params=_SC_PARAMS,
        scratch_types=[
            pltpu.VMEM((32, 128), jnp.int32),
            pltpu.VMEM((32, 128), jnp.int32),
            pltpu.VMEM((128, HID), jnp.float32),
            pltpu.VMEM((128, HID), jnp.float32),
            pltpu.SemaphoreType.DMA,
            pltpu.SemaphoreType.DMA,
        ],
    )(_pedge_body)


def _enc_kernel(x_ref, emb_ref, wn_ref, bn_ref, we_ref, be_ref, h_ref, e_ref):
    h_ref[...] = jnp.maximum(x_ref[...] @ wn_ref[...] + bn_ref[...], 0.0)
    e_ref[...] = jnp.maximum(emb_ref[...] @ we_ref[...] + be_ref[...], 0.0)


def _dinv_kernel(p_ref, o_ref):
    dsum = p_ref[0] + p_ref[1]
    o_ref[...] = jnp.where(dsum > 0.0,
                           lax.rsqrt(jnp.maximum(dsum, 1e-12)), 0.0)


def _pre_kernel(h_ref, dinv_ref, wi_ref, wr_ref, t_ref, r_ref):
    hs = h_ref[...] * dinv_ref[...]
    t_ref[...] = hs @ wi_ref[...]
    r_ref[...] = h_ref[...] @ wr_ref[...]


def _post_kernel(agg_ref, dinv_ref, r_ref, bias_ref, out_ref, sums_ref):
    b = pl.program_id(0)
    o = jnp.maximum(agg_ref[...] * dinv_ref[...] + r_ref[...] + bias_ref[...],
                    0.0)
    out_ref[...] = o
    part = jnp.stack([jnp.sum(o, axis=0), jnp.sum(o * o, axis=0)])

    @pl.when(b == 0)
    def _():
        sums_ref[...] = part

    @pl.when(b > 0)
    def _():
        sums_ref[...] += part


def _bn_kernel(out_ref, sums_ref, gamma_ref, beta_ref, h_ref):
    inv_n = 1.0 / N
    mean = sums_ref[0, :] * inv_n
    var = sums_ref[1, :] * inv_n - mean * mean
    scale = lax.rsqrt(var + 1e-5) * gamma_ref[0]
    h_ref[...] = (out_ref[...] - mean) * scale + beta_ref[0]


def _uv_kernel(h_ref, e_ref, wa_ref, wb_ref, wc_ref, wd_ref, b1_ref,
               u_ref, v_ref):
    u_ref[...] = (h_ref[...] @ wa_ref[...] + e_ref[...] @ wb_ref[...]
                  + b1_ref[...])
    v_ref[...] = h_ref[...] @ wc_ref[...] + e_ref[...] @ wd_ref[...]


def _fin_kernel(u_ref, v_ref, w2_ref, b2_ref, y_ref):
    y_ref[...] = jnp.tanh(u_ref[...] + v_ref[...]) @ w2_ref[...] + b2_ref[0]


def _full(shape):
    nd = len(shape)
    return pl.BlockSpec(shape, lambda b: (0,) * nd)


def kernel(x, emb, edge_index, pedge_index, W_node, b_node, W_emb, b_emb,
           conv_init_w, conv_root_w, conv_bias, bn_gamma, bn_beta,
           W1, b1, W2, b2):
    f32 = jnp.float32
    row = edge_index[0]
    col = edge_index[1]
    row2 = jnp.pad(row, (0, EPAD - E)).reshape(EROWS64, 64)
    col2_pad = jnp.pad(col, (0, EPAD - E), constant_values=JUNK_DEG)
    col2 = col2_pad.reshape(EROWS64, 64)
    col2_deg = col2_pad.reshape(EROWS, 128)
    src2 = jnp.pad(pedge_index[0], (0, PEPAD - PE)).reshape(PEROWS, 128)
    dst2 = jnp.pad(pedge_index[1], (0, PEPAD - PE)).reshape(PEROWS, 128)

    zeros1d = jnp.zeros((3200,), f32)
    zeros2d = jnp.zeros((784, HID), f32)

    Wa = W1[0:72] + W1[144:216] + W1[288:360]
    Wb = W1[72:144] + W1[216:288] + W1[360:432]
    Wc = W1[432:504] + W1[576:648] + W1[720:792]
    Wd = W1[504:576] + W1[648:720] + W1[792:864]

    bn1 = b_node.reshape(1, HID)
    be1 = b_emb.reshape(1, HID)
    b1r = b1.reshape(1, HID)
    b2r = b2.reshape(1, 1)

    deg_parts = _make_deg_kernel()(col2_deg, zeros1d)
    selr1d, sellc1d, counts = _make_part_kernel()(row2, col2)
    selr = selr1d.reshape(4, 32, CAPC, 128)
    sellc = sellc1d.reshape(4, 32, CAPC, 128)

    grid25 = 25
    BLK = 2000
    h, e = pl.pallas_call(
        _enc_kernel,
        grid=(grid25,),
        in_specs=[
            pl.BlockSpec((BLK, D), lambda b: (b, 0)),
            pl.BlockSpec((BLK, D), lambda b: (b, 0)),
            _full((D, HID)), _full((1, HID)),
            _full((D, HID)), _full((1, HID)),
        ],
        out_specs=[
            pl.BlockSpec((BLK, HID), lambda b: (b, 0)),
            pl.BlockSpec((BLK, HID), lambda b: (b, 0)),
        ],
        out_shape=[
            jax.ShapeDtypeStruct((N, HID), f32),
            jax.ShapeDtypeStruct((N, HID), f32),
        ],
    )(x, emb, W_node, bn1, W_emb, be1)

    dinv2d = pl.pallas_call(
        _dinv_kernel,
        out_shape=jax.ShapeDtypeStruct((400, 128), f32),
    )(deg_parts.reshape(2, 400, 128))
    dinv = dinv2d.reshape(DEG_ROWS, 1)[:N]

    spmm = _make_spmm_kernel()

    for l in range(3):
        wi = conv_init_w[l]
        wr = conv_root_w[l]
        bias = conv_bias[l].reshape(1, HID)
        t, r = pl.pallas_call(
            _pre_kernel,
            grid=(grid25,),
            in_specs=[
                pl.BlockSpec((BLK, HID), lambda b: (b, 0)),
                pl.BlockSpec((BLK, 1), lambda b: (b, 0)),
                _full((HID, HID)), _full((HID, HID)),
            ],
            out_specs=[
                pl.BlockSpec((BLK, HID), lambda b: (b, 0)),
                pl.BlockSpec((BLK, HID), lambda b: (b, 0)),
            ],
            out_shape=[
                jax.ShapeDtypeStruct((N, HID), f32),
                jax.ShapeDtypeStruct((N, HID), f32),
            ],
        )(h, dinv, wi, wr)

        agg_parts = spmm(t, selr, sellc, counts, zeros2d)
        agg0 = agg_parts[:, :QUARTER, :].reshape(N, HID)

        out, sums = pl.pallas_call(
            _post_kernel,
            grid=(50,),
            in_specs=[
                pl.BlockSpec((1000, HID), lambda b: (b, 0)),
                pl.BlockSpec((1000, 1), lambda b: (b, 0)),
                pl.BlockSpec((1000, HID), lambda b: (b, 0)),
                _full((1, HID)),
            ],
            out_specs=[
                pl.BlockSpec((1000, HID), lambda b: (b, 0)),
                pl.BlockSpec((2, HID), lambda b: (0, 0)),
            ],
            out_shape=[
                jax.ShapeDtypeStruct((N, HID), f32),
                jax.ShapeDtypeStruct((2, HID), f32),
            ],
        )(agg0, dinv, r, bias)

        h = pl.pallas_call(
            _bn_kernel,
            grid=(grid25,),
            in_specs=[
                pl.BlockSpec((BLK, HID), lambda b: (b, 0)),
                _full((2, HID)), _full((1, HID)), _full((1, HID)),
            ],
            out_specs=pl.BlockSpec((BLK, HID), lambda b: (b, 0)),
            out_shape=jax.ShapeDtypeStruct((N, HID), f32),
        )(out, sums, bn_gamma[l].reshape(1, HID), bn_beta[l].reshape(1, HID))

    u, v = pl.pallas_call(
        _uv_kernel,
        grid=(grid25,),
        in_specs=[
            pl.BlockSpec((BLK, HID), lambda b: (b, 0)),
            pl.BlockSpec((BLK, HID), lambda b: (b, 0)),
            _full((HID, HID)), _full((HID, HID)),
            _full((HID, HID)), _full((HID, HID)),
            _full((1, HID)),
        ],
        out_specs=[
            pl.BlockSpec((BLK, HID), lambda b: (b, 0)),
            pl.BlockSpec((BLK, HID), lambda b: (b, 0)),
        ],
        out_shape=[
            jax.ShapeDtypeStruct((N, HID), f32),
            jax.ShapeDtypeStruct((N, HID), f32),
        ],
    )(h, e, Wa, Wb, Wc, Wd, b1r)

    U, V = _make_pedge_kernel()(u, v, src2, dst2)

    y = pl.pallas_call(
        _fin_kernel,
        grid=(32,),
        in_specs=[
            pl.BlockSpec((4096, HID), lambda b: (b, 0)),
            pl.BlockSpec((4096, HID), lambda b: (b, 0)),
            _full((HID, 1)), _full((1, 1)),
        ],
        out_specs=pl.BlockSpec((4096, 1), lambda b: (b, 0)),
        out_shape=jax.ShapeDtypeStruct((PEPAD, 1), f32),
    )(U, V, W2, b2r)

    return y[:PE]
